# baseline (speedup 1.0000x reference)
import functools

import jax
import jax.numpy as jnp
from jax import lax
from jax.experimental import pallas as pl
from jax.experimental.pallas import tpu as pltpu
from jax.experimental.pallas import tpu_sc as plsc


def _make_gather(num_rows: int, dim: int, batch: int):
    info = plsc.get_sparse_core_info()
    nc, ns = info.num_cores, info.num_subcores
    nw = nc * ns
    assert batch % (8 * nw) == 0
    b_per_w = batch // nw
    mesh = plsc.VectorSubcoreMesh(core_axis_name="c", subcore_axis_name="s")

    n_chunks = 4
    chunk = b_per_w // n_chunks

    @functools.partial(
        pl.kernel,
        mesh=mesh,
        compiler_params=pltpu.CompilerParams(use_tc_tiling_on_sc=False),
        out_type=jax.ShapeDtypeStruct((batch, dim), jnp.float32),
        scratch_types=[
            pltpu.VMEM((b_per_w,), jnp.int32),
            pltpu.VMEM((2, chunk, dim), jnp.float32),
            [pltpu.SemaphoreType.DMA] * 2,
            [pltpu.SemaphoreType.DMA] * 2,
        ],
    )
    def gather_kernel(table_hbm, idx_hbm, out_hbm, idx_v, rows_v, gsem, ssem):
        wid = lax.axis_index("s") * nc + lax.axis_index("c")
        base = wid * b_per_w
        pltpu.sync_copy(idx_hbm.at[pl.ds(base, b_per_w)], idx_v)

        def start_gather(c):
            b = c % 2
            pltpu.async_copy(
                table_hbm.at[idx_v.at[pl.ds(c * chunk, chunk)]],
                rows_v.at[b],
                gsem[b],
            )

        start_gather(0)
        for c in range(n_chunks):
            b = c % 2
            if c + 1 < n_chunks:
                if c + 1 >= 2:
                    pltpu.make_async_copy(
                        rows_v.at[(c + 1) % 2],
                        out_hbm.at[pl.ds(base + (c - 1) * chunk, chunk)],
                        ssem[(c + 1) % 2],
                    ).wait()
                start_gather(c + 1)
            pltpu.make_async_copy(
                table_hbm.at[idx_v.at[pl.ds(c * chunk, chunk)]],
                rows_v.at[b],
                gsem[b],
            ).wait()
            pltpu.async_copy(
                rows_v.at[b],
                out_hbm.at[pl.ds(base + c * chunk, chunk)],
                ssem[b],
            )
        for c in range(n_chunks - 2, n_chunks):
            b = c % 2
            pltpu.make_async_copy(
                rows_v.at[b],
                out_hbm.at[pl.ds(base + c * chunk, chunk)],
                ssem[b],
            ).wait()

    return gather_kernel


def kernel(embed_id, weight):
    num_rows, dim = weight.shape
    batch = embed_id.size
    idx_flat = embed_id.reshape(-1).astype(jnp.int32)
    out = _make_gather(num_rows, dim, batch)(weight, idx_flat)
    return out.reshape(embed_id.shape + (dim,))


# speedup vs baseline: 1.0159x; 1.0159x over previous
import functools

import jax
import jax.numpy as jnp
from jax import lax
from jax.experimental import pallas as pl
from jax.experimental.pallas import tpu as pltpu
from jax.experimental.pallas import tpu_sc as plsc


def _make_gather(num_ids_rows: int, num_ids_cols: int, dim: int):
    info = plsc.get_sparse_core_info()
    nc, ns = info.num_cores, info.num_subcores
    nw = nc * ns
    batch = num_ids_rows * num_ids_cols
    assert batch % (8 * nw) == 0
    b_per_w = batch // nw
    assert num_ids_cols % b_per_w == 0 or b_per_w % num_ids_cols == 0
    per_row = num_ids_cols // b_per_w
    mesh = plsc.VectorSubcoreMesh(core_axis_name="c", subcore_axis_name="s")

    @functools.partial(
        pl.kernel,
        mesh=mesh,
        compiler_params=pltpu.CompilerParams(use_tc_tiling_on_sc=False),
        out_type=jax.ShapeDtypeStruct((num_ids_rows, num_ids_cols, dim), jnp.float32),
        scratch_types=[
            pltpu.VMEM((b_per_w,), jnp.int32),
            pltpu.VMEM((b_per_w, dim), jnp.float32),
            pltpu.SemaphoreType.DMA,
        ],
    )
    def gather_kernel(table_hbm, idx_hbm, out_hbm, idx_v, rows_v, sem):
        wid = lax.axis_index("s") * nc + lax.axis_index("c")
        r = wid // per_row
        col = (wid % per_row) * b_per_w
        pltpu.sync_copy(idx_hbm.at[r, pl.ds(col, b_per_w)], idx_v)
        pltpu.async_copy(table_hbm.at[idx_v], rows_v, sem).wait()
        pltpu.sync_copy(rows_v, out_hbm.at[r, pl.ds(col, b_per_w)])

    return gather_kernel


def kernel(embed_id, weight):
    num_rows, dim = weight.shape
    ir, ic = embed_id.shape
    out = _make_gather(ir, ic, dim)(weight, embed_id.astype(jnp.int32))
    return out
